# Initial kernel scaffold; baseline (speedup 1.0000x reference)
import jax
import jax.numpy as jnp
from jax.experimental import pallas as pl


def kernel(query, prompt_pool, prompt_keys):
    raise NotImplementedError("write your pallas kernel here")


# speedup vs baseline: 1.7949x; 1.7949x over previous
import functools

import jax
import jax.numpy as jnp
from jax import lax
from jax.experimental import pallas as pl
from jax.experimental.pallas import tpu as pltpu
from jax.experimental.pallas import tpu_sc as plsc

_POOL = 256
_LP = 8
_D = 768
_H = 12
_L = 6
_K = 4
_DH = 64
_B = 128

_ROWS_PER_PROMPT = _L * 2 * _LP * _H
_N_OUT_ROWS = 2 * _L * _B * _H * _K * _LP
_CHUNK = 128
_N_CHUNKS = _N_OUT_ROWS // _CHUNK

_NC = 2
_NS = 16
_NW = _NC * _NS
_CPW = _N_CHUNKS // _NW
_G = 2
_ROUNDS = _CPW // _G


def _select_and_index_body(q_ref, k_ref, s_ref):
    q = q_ref[...]
    k = k_ref[...]
    qn = q / jnp.maximum(jnp.sqrt(jnp.sum(q * q, axis=1, keepdims=True)), 1e-12)
    kn = k / jnp.maximum(jnp.sqrt(jnp.sum(k * k, axis=1, keepdims=True)), 1e-12)
    sims = lax.dot_general(
        qn, kn,
        dimension_numbers=(((1,), (1,)), ((), ())),
        precision=lax.Precision.DEFAULT,
        preferred_element_type=jnp.float32,
    )

    col = lax.broadcasted_iota(jnp.int32, (_B, _POOL), 1)
    sels = []
    cur = sims
    for _ in range(_K):
        m = jnp.max(cur, axis=1, keepdims=True)
        cand = jnp.where(cur == m, col, jnp.int32(2 ** 30))
        sel = jnp.min(cand, axis=1, keepdims=True)
        sels.append(sel)
        cur = jnp.where(col == sel, -jnp.inf, cur)

    u = lax.broadcasted_iota(jnp.int32, (_B, _H * _K * _LP), 1)
    h = u // 32
    s = u - h * 32
    t = s // 8
    p = s - t * 8
    selv = jnp.where(t == 0, sels[0],
                     jnp.where(t == 1, sels[1],
                               jnp.where(t == 2, sels[2], sels[3])))
    rows = selv * _ROWS_PER_PROMPT + p * _H + h
    c_i = lax.broadcasted_iota(jnp.int32, (2, _L, _B, _H * _K * _LP), 0)
    l_i = lax.broadcasted_iota(jnp.int32, (2, _L, _B, _H * _K * _LP), 1)
    s_ref[...] = (jnp.broadcast_to(rows, (2, _L, _B, _H * _K * _LP))
                  + (l_i * 2 + c_i) * (_LP * _H))


_select_and_index = pl.pallas_call(
    _select_and_index_body,
    out_shape=jax.ShapeDtypeStruct((2, _L, _B, _H * _K * _LP), jnp.int32),
)


def _gather_body(s_hbm, table_hbm, out_hbm, idx_v, bufs, gsem0, gsem1,
                 wsem0, wsem1):
    wid = lax.axis_index("subcore") * _NC + lax.axis_index("core")
    base = wid * _CPW
    pltpu.sync_copy(s_hbm.at[pl.ds(base, _CPW)], idx_v)

    gsems = (gsem0, gsem1)
    wsems = (wsem0, wsem1)

    def start_gathers(r, st):
        for g in range(_G):
            c = r * _G + g
            pltpu.async_copy(table_hbm.at[idx_v.at[c]], bufs.at[st, g],
                             gsems[st])

    def wait_gathers(r, st):
        for g in range(_G):
            c = r * _G + g
            pltpu.make_async_copy(table_hbm.at[idx_v.at[c]], bufs.at[st, g],
                                  gsems[st]).wait()

    def start_writes(r, st):
        for g in range(_G):
            n = base + r * _G + g
            pltpu.async_copy(bufs.at[st, g], out_hbm.at[n], wsems[st])

    def wait_writes(r, st):
        for g in range(_G):
            n = base + r * _G + g
            pltpu.make_async_copy(bufs.at[st, g], out_hbm.at[n],
                                  wsems[st]).wait()

    start_gathers(0, 0)

    @pl.loop(0, _ROUNDS // 2)
    def _(r2):
        ra = 2 * r2
        rb = ra + 1
        wait_gathers(ra, 0)

        @pl.when(r2 > 0)
        def _():
            wait_writes(ra - 1, 1)

        start_gathers(rb, 1)
        start_writes(ra, 0)

        wait_gathers(rb, 1)
        wait_writes(ra, 0)

        @pl.when(r2 + 1 < _ROUNDS // 2)
        def _():
            start_gathers(ra + 2, 0)

        start_writes(rb, 1)

    wait_writes(_ROUNDS - 1, 1)


@functools.cache
def _make_gather():
    return pl.kernel(
        _gather_body,
        out_type=jax.ShapeDtypeStruct((_N_CHUNKS, _CHUNK, _DH), jnp.float32),
        mesh=plsc.VectorSubcoreMesh(core_axis_name="core",
                                    subcore_axis_name="subcore",
                                    num_cores=_NC, num_subcores=_NS),
        scratch_types=[
            pltpu.VMEM((_CPW, _CHUNK), jnp.int32),
            pltpu.VMEM((2, _G, _CHUNK, _DH), jnp.float32),
            pltpu.SemaphoreType.DMA,
            pltpu.SemaphoreType.DMA,
            pltpu.SemaphoreType.DMA,
            pltpu.SemaphoreType.DMA,
        ],
        compiler_params=pltpu.CompilerParams(use_tc_tiling_on_sc=False),
    )


def kernel(query, prompt_pool, prompt_keys):
    src_rows = _select_and_index(query, prompt_keys)
    src_rows = src_rows.reshape(_N_CHUNKS, _CHUNK)
    table = prompt_pool.reshape(_POOL * _ROWS_PER_PROMPT, _DH)
    out = _make_gather()(src_rows, table)
    return out.reshape(2, _L, _B, _H, _K * _LP, _DH)


# speedup vs baseline: 3.5350x; 1.9695x over previous
import functools

import jax
import jax.numpy as jnp
from jax import lax
from jax.experimental import pallas as pl
from jax.experimental.pallas import tpu as pltpu
from jax.experimental.pallas import tpu_sc as plsc

_POOL = 256
_LP = 8
_D = 768
_H = 12
_L = 6
_K = 4
_DH = 64
_B = 128

_NC = 2
_NS = 16
_NW = _NC * _NS

_NBU = _L * 2 * _LP * _H
_BUPW = _NBU // _NW
_BLK = _DH * _POOL
_TILE = _DH * _B
_NT = 2 * _L * _H * _K * _LP


def _select_g_body(q_ref, k_ref, g_ref):
    q = q_ref[...]
    k = k_ref[...]
    qn = q / jnp.maximum(jnp.sqrt(jnp.sum(q * q, axis=1, keepdims=True)), 1e-12)
    kn = k / jnp.maximum(jnp.sqrt(jnp.sum(k * k, axis=1, keepdims=True)), 1e-12)
    sims = lax.dot_general(
        qn, kn,
        dimension_numbers=(((1,), (1,)), ((), ())),
        precision=lax.Precision.DEFAULT,
        preferred_element_type=jnp.float32,
    )

    col = lax.broadcasted_iota(jnp.int32, (_B, _POOL), 1)
    sels = []
    cur = sims
    for _ in range(_K):
        m = jnp.max(cur, axis=1, keepdims=True)
        cand = jnp.where(cur == m, col, jnp.int32(2 ** 30))
        sel = jnp.min(cand, axis=1, keepdims=True)
        sels.append(sel)
        cur = jnp.where(col == sel, -jnp.inf, cur)

    idx = jnp.concatenate(sels, axis=1)
    g = (idx // 128) * 1024 + idx % 128
    gt = jnp.swapaxes(g, 0, 1)
    g_ref[...] = jnp.concatenate(
        [gt, jnp.zeros((4, _B), jnp.int32)], axis=0)


_select_g = pl.pallas_call(
    _select_g_body,
    out_shape=jax.ShapeDtypeStruct((8, _B), jnp.int32),
)


def _block_row(bu):
    c = bu // (_L * _H * _LP)
    l = (bu // (_H * _LP)) % _L
    h = (bu // _LP) % _H
    p = bu % _LP
    return ((l * 2 + c) * _LP + p) * _H + h


def _tile_row(bu, t):
    return (bu // _LP) * 32 + t * 8 + bu % _LP


def _gather_body_v2(g_hbm, v_hbm, out_hbm, g_v, blk0, blk1, tl0, tl1,
                    bsem0, bsem1, wsem0, wsem1):
    wid = lax.axis_index("subcore") * _NC + lax.axis_index("core")
    base = wid * _BUPW
    pltpu.sync_copy(g_hbm, g_v)

    def start_fetch(uu, bref, sem):
        pltpu.async_copy(v_hbm.at[_block_row(base + uu)], bref, sem)

    def wait_fetch(uu, bref, sem):
        pltpu.make_async_copy(v_hbm.at[_block_row(base + uu)], bref,
                              sem).wait()

    def compute(bref, tref):
        @pl.loop(0, 8)
        def _(kk):
            for t in range(_K):
                gv = g_v[t, pl.ds(kk * 16, 16)]
                for d in range(_DH):
                    off = (d // 8) * 2048 + (d % 8) * 128
                    val = plsc.load_gather(bref, [gv + off])
                    tref[t, pl.ds(d * 128 + kk * 16, 16)] = val

    def start_writes(uu, tref, sem):
        for t in range(_K):
            pltpu.async_copy(tref.at[t], out_hbm.at[_tile_row(base + uu, t)],
                             sem)

    def wait_writes(uu, tref, sem):
        for t in range(_K):
            pltpu.make_async_copy(tref.at[t],
                                  out_hbm.at[_tile_row(base + uu, t)],
                                  sem).wait()

    start_fetch(0, blk0, bsem0)

    @pl.loop(0, _BUPW // 2)
    def _(u2):
        ua = 2 * u2
        ub = ua + 1

        wait_fetch(ua, blk0, bsem0)
        start_fetch(ub, blk1, bsem1)

        @pl.when(u2 > 0)
        def _():
            wait_writes(ua - 2, tl0, wsem0)

        compute(blk0, tl0)
        start_writes(ua, tl0, wsem0)

        wait_fetch(ub, blk1, bsem1)

        @pl.when(u2 + 1 < _BUPW // 2)
        def _():
            start_fetch(ua + 2, blk0, bsem0)

        @pl.when(u2 > 0)
        def _():
            wait_writes(ub - 2, tl1, wsem1)

        compute(blk1, tl1)
        start_writes(ub, tl1, wsem1)

    wait_writes(_BUPW - 2, tl0, wsem0)
    wait_writes(_BUPW - 1, tl1, wsem1)


@functools.cache
def _make_gather():
    return pl.kernel(
        _gather_body_v2,
        out_type=jax.ShapeDtypeStruct((_NT, _TILE), jnp.float32),
        mesh=plsc.VectorSubcoreMesh(core_axis_name="core",
                                    subcore_axis_name="subcore",
                                    num_cores=_NC, num_subcores=_NS),
        scratch_types=[
            pltpu.VMEM((8, _B), jnp.int32),
            pltpu.VMEM((_BLK,), jnp.float32),
            pltpu.VMEM((_BLK,), jnp.float32),
            pltpu.VMEM((_K, _TILE), jnp.float32),
            pltpu.VMEM((_K, _TILE), jnp.float32),
            pltpu.SemaphoreType.DMA,
            pltpu.SemaphoreType.DMA,
            pltpu.SemaphoreType.DMA,
            pltpu.SemaphoreType.DMA,
        ],
        compiler_params=pltpu.CompilerParams(use_tc_tiling_on_sc=False,
                                             needs_layout_passes=False),
    )


def kernel(query, prompt_pool, prompt_keys):
    g8 = _select_g(query, prompt_keys)
    v = (prompt_pool
         .transpose(1, 2, 3, 4, 5, 0)
         .reshape(_L, 2, _LP, _H, 8, 8, 2, 128)
         .transpose(0, 1, 2, 3, 4, 6, 5, 7)
         .reshape(_NBU, _BLK))
    out = _make_gather()(g8, v)
    return (out.reshape(2, _L, _H, _K * _LP, _DH, _B)
               .transpose(0, 1, 5, 2, 3, 4))


# speedup vs baseline: 11.7078x; 3.3120x over previous
import functools

import jax
import jax.numpy as jnp
from jax import lax
from jax.experimental import pallas as pl
from jax.experimental.pallas import tpu as pltpu
from jax.experimental.pallas import tpu_sc as plsc

_POOL = 256
_LP = 8
_D = 768
_H = 12
_L = 6
_K = 4
_DH = 64
_B = 128

_NC = 2
_NS = 16
_NW = _NC * _NS

_NBU = _L * 2 * _LP * _H
_BUPW = _NBU // _NW
_BLK = _DH * _POOL
_TILE = _DH * _B
_NT = 2 * _L * _H * _K * _LP


def _select_g_body(q_ref, k_ref, g_ref):
    q = q_ref[...]
    k = k_ref[...]
    qn = q / jnp.maximum(jnp.sqrt(jnp.sum(q * q, axis=1, keepdims=True)), 1e-12)
    kn = k / jnp.maximum(jnp.sqrt(jnp.sum(k * k, axis=1, keepdims=True)), 1e-12)
    sims = lax.dot_general(
        qn, kn,
        dimension_numbers=(((1,), (1,)), ((), ())),
        precision=lax.Precision.DEFAULT,
        preferred_element_type=jnp.float32,
    )

    col = lax.broadcasted_iota(jnp.int32, (_B, _POOL), 1)
    sels = []
    cur = sims
    for _ in range(_K):
        m = jnp.max(cur, axis=1, keepdims=True)
        cand = jnp.where(cur == m, col, jnp.int32(2 ** 30))
        sel = jnp.min(cand, axis=1, keepdims=True)
        sels.append(sel)
        cur = jnp.where(col == sel, -jnp.inf, cur)

    idx = jnp.concatenate(sels, axis=1)
    g = (idx // 128) * 1024 + idx % 128
    gt = jnp.swapaxes(g, 0, 1)
    g_ref[...] = jnp.concatenate(
        [gt, jnp.zeros((4, _B), jnp.int32)], axis=0)


_select_g = pl.pallas_call(
    _select_g_body,
    out_shape=jax.ShapeDtypeStruct((8, _B), jnp.int32),
)


def _block_row(bu):
    c = bu // (_L * _H * _LP)
    l = (bu // (_H * _LP)) % _L
    h = (bu // _LP) % _H
    p = bu % _LP
    return ((l * 2 + c) * _LP + p) * _H + h


def _tile_row(bu, t):
    return (bu // _LP) * 32 + t * 8 + bu % _LP


def _gather_body_v2(g_hbm, v_hbm, out_hbm, g_v, blk0, blk1, tl0, tl1,
                    bsem0, bsem1, wsem0, wsem1):
    wid = lax.axis_index("subcore") * _NC + lax.axis_index("core")
    base = wid * _BUPW
    pltpu.sync_copy(g_hbm, g_v)

    def start_fetch(uu, bref, sem):
        pltpu.async_copy(v_hbm.at[_block_row(base + uu)], bref, sem)

    def wait_fetch(uu, bref, sem):
        pltpu.make_async_copy(v_hbm.at[_block_row(base + uu)], bref,
                              sem).wait()

    def compute(bref, tref):
        @functools.partial(plsc.parallel_loop, 0, 8, unroll=2)
        def _(kk):
            for t in range(_K):
                gv = g_v[t, pl.ds(kk * 16, 16)]
                for d8 in range(8):
                    vals = []
                    for din in range(8):
                        off = d8 * 2048 + din * 128
                        vals.append(plsc.load_gather(bref, [gv + off]))
                    for din in range(8):
                        d = d8 * 8 + din
                        tref[t, pl.ds(d * 128 + kk * 16, 16)] = vals[din]

    def start_writes(uu, tref, sem):
        for t in range(_K):
            pltpu.async_copy(tref.at[t], out_hbm.at[_tile_row(base + uu, t)],
                             sem)

    def wait_writes(uu, tref, sem):
        for t in range(_K):
            pltpu.make_async_copy(tref.at[t],
                                  out_hbm.at[_tile_row(base + uu, t)],
                                  sem).wait()

    start_fetch(0, blk0, bsem0)

    @pl.loop(0, _BUPW // 2)
    def _(u2):
        ua = 2 * u2
        ub = ua + 1

        wait_fetch(ua, blk0, bsem0)
        start_fetch(ub, blk1, bsem1)

        @pl.when(u2 > 0)
        def _():
            wait_writes(ua - 2, tl0, wsem0)

        compute(blk0, tl0)
        start_writes(ua, tl0, wsem0)

        wait_fetch(ub, blk1, bsem1)

        @pl.when(u2 + 1 < _BUPW // 2)
        def _():
            start_fetch(ua + 2, blk0, bsem0)

        @pl.when(u2 > 0)
        def _():
            wait_writes(ub - 2, tl1, wsem1)

        compute(blk1, tl1)
        start_writes(ub, tl1, wsem1)

    wait_writes(_BUPW - 2, tl0, wsem0)
    wait_writes(_BUPW - 1, tl1, wsem1)


@functools.cache
def _make_gather():
    return pl.kernel(
        _gather_body_v2,
        out_type=jax.ShapeDtypeStruct((_NT, _TILE), jnp.float32),
        mesh=plsc.VectorSubcoreMesh(core_axis_name="core",
                                    subcore_axis_name="subcore",
                                    num_cores=_NC, num_subcores=_NS),
        scratch_types=[
            pltpu.VMEM((8, _B), jnp.int32),
            pltpu.VMEM((_BLK,), jnp.float32),
            pltpu.VMEM((_BLK,), jnp.float32),
            pltpu.VMEM((_K, _TILE), jnp.float32),
            pltpu.VMEM((_K, _TILE), jnp.float32),
            pltpu.SemaphoreType.DMA,
            pltpu.SemaphoreType.DMA,
            pltpu.SemaphoreType.DMA,
            pltpu.SemaphoreType.DMA,
        ],
        compiler_params=pltpu.CompilerParams(use_tc_tiling_on_sc=False,
                                             needs_layout_passes=False),
    )


def kernel(query, prompt_pool, prompt_keys):
    g8 = _select_g(query, prompt_keys)
    v = (prompt_pool
         .transpose(1, 2, 3, 4, 5, 0)
         .reshape(_L, 2, _LP, _H, 8, 8, 2, 128)
         .transpose(0, 1, 2, 3, 4, 6, 5, 7)
         .reshape(_NBU, _BLK))
    out = _make_gather()(g8, v)
    return (out.reshape(2, _L, _H, _K * _LP, _DH, _B)
               .transpose(0, 1, 5, 2, 3, 4))


# speedup vs baseline: 11.7213x; 1.0011x over previous
import functools

import jax
import jax.numpy as jnp
from jax import lax
from jax.experimental import pallas as pl
from jax.experimental.pallas import tpu as pltpu
from jax.experimental.pallas import tpu_sc as plsc

_POOL = 256
_LP = 8
_D = 768
_H = 12
_L = 6
_K = 4
_DH = 64
_B = 128

_NC = 2
_NS = 16
_NW = _NC * _NS

_NBU = _L * 2 * _LP * _H
_BUPW = _NBU // _NW
_BLK = _DH * _POOL
_TILE = _DH * _B
_NT = 2 * _L * _H * _K * _LP


def _select_g_body(q_ref, k_ref, g_ref):
    q = q_ref[...]
    k = k_ref[...]
    qn = q / jnp.maximum(jnp.sqrt(jnp.sum(q * q, axis=1, keepdims=True)), 1e-12)
    kn = k / jnp.maximum(jnp.sqrt(jnp.sum(k * k, axis=1, keepdims=True)), 1e-12)
    sims = lax.dot_general(
        qn, kn,
        dimension_numbers=(((1,), (1,)), ((), ())),
        precision=lax.Precision.DEFAULT,
        preferred_element_type=jnp.float32,
    )

    col = lax.broadcasted_iota(jnp.int32, (_B, _POOL), 1)
    sels = []
    cur = sims
    for _ in range(_K):
        m = jnp.max(cur, axis=1, keepdims=True)
        cand = jnp.where(cur == m, col, jnp.int32(2 ** 30))
        sel = jnp.min(cand, axis=1, keepdims=True)
        sels.append(sel)
        cur = jnp.where(col == sel, -jnp.inf, cur)

    idx = jnp.concatenate(sels, axis=1)
    g = (idx // 128) * 1024 + idx % 128
    gt = jnp.swapaxes(g, 0, 1)
    g_ref[...] = jnp.concatenate(
        [gt, jnp.zeros((4, _B), jnp.int32)], axis=0)


_select_g = pl.pallas_call(
    _select_g_body,
    out_shape=jax.ShapeDtypeStruct((8, _B), jnp.int32),
)


def _block_row(bu):
    c = bu // (_L * _H * _LP)
    l = (bu // (_H * _LP)) % _L
    h = (bu // _LP) % _H
    p = bu % _LP
    return ((l * 2 + c) * _LP + p) * _H + h


def _tile_row(bu, t):
    return (bu // _LP) * 32 + t * 8 + bu % _LP


def _gather_body_v2(g_hbm, v_hbm, out_hbm, g_v, blk0, blk1, tl0, tl1,
                    bsem0, bsem1, wsem0, wsem1):
    wid = lax.axis_index("subcore") * _NC + lax.axis_index("core")
    base = wid * _BUPW
    pltpu.sync_copy(g_hbm, g_v)

    def start_fetch(uu, bref, sem):
        pltpu.async_copy(v_hbm.at[_block_row(base + uu)], bref, sem)

    def wait_fetch(uu, bref, sem):
        pltpu.make_async_copy(v_hbm.at[_block_row(base + uu)], bref,
                              sem).wait()

    def compute(bref, tref):
        @functools.partial(plsc.parallel_loop, 0, 8)
        def _(kk):
            for t in range(_K):
                gv = g_v[t, pl.ds(kk * 16, 16)]
                for d8 in range(8):
                    vals = []
                    for din in range(8):
                        off = d8 * 2048 + din * 128
                        vals.append(plsc.load_gather(bref, [gv + off]))
                    for din in range(8):
                        d = d8 * 8 + din
                        tref[t, pl.ds(d * 128 + kk * 16, 16)] = vals[din]

    def start_writes(uu, tref, sem):
        for t in range(_K):
            pltpu.async_copy(tref.at[t], out_hbm.at[_tile_row(base + uu, t)],
                             sem)

    def wait_writes(uu, tref, sem):
        for t in range(_K):
            pltpu.make_async_copy(tref.at[t],
                                  out_hbm.at[_tile_row(base + uu, t)],
                                  sem).wait()

    start_fetch(0, blk0, bsem0)

    @pl.loop(0, _BUPW // 2)
    def _(u2):
        ua = 2 * u2
        ub = ua + 1

        wait_fetch(ua, blk0, bsem0)
        start_fetch(ub, blk1, bsem1)

        @pl.when(u2 > 0)
        def _():
            wait_writes(ua - 2, tl0, wsem0)

        compute(blk0, tl0)
        start_writes(ua, tl0, wsem0)

        wait_fetch(ub, blk1, bsem1)

        @pl.when(u2 + 1 < _BUPW // 2)
        def _():
            start_fetch(ua + 2, blk0, bsem0)

        @pl.when(u2 > 0)
        def _():
            wait_writes(ub - 2, tl1, wsem1)

        compute(blk1, tl1)
        start_writes(ub, tl1, wsem1)

    wait_writes(_BUPW - 2, tl0, wsem0)
    wait_writes(_BUPW - 1, tl1, wsem1)


@functools.cache
def _make_gather():
    return pl.kernel(
        _gather_body_v2,
        out_type=jax.ShapeDtypeStruct((_NT, _TILE), jnp.float32),
        mesh=plsc.VectorSubcoreMesh(core_axis_name="core",
                                    subcore_axis_name="subcore",
                                    num_cores=_NC, num_subcores=_NS),
        scratch_types=[
            pltpu.VMEM((8, _B), jnp.int32),
            pltpu.VMEM((_BLK,), jnp.float32),
            pltpu.VMEM((_BLK,), jnp.float32),
            pltpu.VMEM((_K, _TILE), jnp.float32),
            pltpu.VMEM((_K, _TILE), jnp.float32),
            pltpu.SemaphoreType.DMA,
            pltpu.SemaphoreType.DMA,
            pltpu.SemaphoreType.DMA,
            pltpu.SemaphoreType.DMA,
        ],
        compiler_params=pltpu.CompilerParams(use_tc_tiling_on_sc=False,
                                             needs_layout_passes=False),
    )


def kernel(query, prompt_pool, prompt_keys):
    g8 = _select_g(query, prompt_keys)
    v = (prompt_pool
         .transpose(1, 2, 3, 4, 5, 0)
         .reshape(_L, 2, _LP, _H, 8, 8, 2, 128)
         .transpose(0, 1, 2, 3, 4, 6, 5, 7)
         .reshape(_NBU, _BLK))
    out = _make_gather()(g8, v)
    return (out.reshape(2, _L, _H, _K * _LP, _DH, _B)
               .transpose(0, 1, 5, 2, 3, 4))
